# baseline (speedup 1.0000x reference)
import functools

import jax
import jax.numpy as jnp
from jax import lax
from jax.experimental import pallas as pl
from jax.experimental.pallas import tpu as pltpu, tpu_sc as plsc

NUM_EMBEDDINGS = 1000000
EMBEDDING_DIM = 64
BATCH = 4
SEQ_LEN = 8192

_INFO = plsc.get_sparse_core_info()
_NC, _NS = _INFO.num_cores, _INFO.num_subcores
_NW = _NC * _NS
_B = BATCH * SEQ_LEN
_B_PER_W = _B // _NW
_W = 128
_NWIN = _B_PER_W // _W
_PAIR_ROWS = 500224
_PW = 2 * EMBEDDING_DIM
_HALF = _PAIR_ROWS


def _make_gather():
    mesh = plsc.VectorSubcoreMesh(core_axis_name="c", subcore_axis_name="s")

    @functools.partial(
        pl.kernel,
        mesh=mesh,
        out_type=jax.ShapeDtypeStruct((_B, EMBEDDING_DIM), jnp.float32),
        scratch_types=[
            pltpu.VMEM((_B_PER_W,), jnp.int32),
            pltpu.VMEM((_B_PER_W,), jnp.int32),
            pltpu.VMEM((2, _W, _PW), jnp.float32),
            pltpu.VMEM((2, _W, EMBEDDING_DIM), jnp.float32),
            pltpu.SemaphoreType.DMA,
            pltpu.SemaphoreType.DMA,
            pltpu.SemaphoreType.DMA,
            pltpu.SemaphoreType.DMA,
        ],
        compiler_params=pltpu.CompilerParams(needs_layout_passes=False),
    )
    def gather_kernel(
        table_hbm, idx_hbm, out_hbm, idx_v, pair_v, rows_v, sel_v, g0, g1, s0, s1
    ):
        wid = lax.axis_index("s") * _NC + lax.axis_index("c")
        base = wid * _B_PER_W
        gsem = (g0, g1)
        ssem = (s0, s1)
        pltpu.sync_copy(idx_hbm.at[pl.ds(base, _B_PER_W)], idx_v)

        def pair_body(k, _):
            o = pl.multiple_of(k * 16, 16)
            ids = idx_v[pl.ds(o, 16)]
            hi = (ids >= _HALF).astype(jnp.int32)
            pair_v[pl.ds(o, 16)] = ids - hi * _HALF
            return _

        lax.fori_loop(0, _B_PER_W // 16, pair_body, None)

        def gather_desc(w, p):
            src = table_hbm.at[pair_v.at[pl.ds(pl.multiple_of(w * _W, _W), _W)]]
            return pltpu.make_async_copy(src, rows_v.at[p], gsem[p])

        def scatter_desc(w, p):
            dst = out_hbm.at[pl.ds(pl.multiple_of(base + w * _W, _W), _W)]
            return pltpu.make_async_copy(sel_v.at[p], dst, ssem[p])

        def select(w, p):
            for g in range(_W // 16):
                o = pl.multiple_of(w * _W + g * 16, 16)
                row16 = lax.iota(jnp.int32, 16) + g * 16
                off16 = (idx_v[pl.ds(o, 16)] >= _HALF).astype(
                    jnp.int32
                ) * EMBEDDING_DIM

                def col_body(j, _):
                    vals = plsc.load_gather(rows_v.at[p], [row16, off16 + j])
                    plsc.store_scatter(
                        sel_v.at[p],
                        [row16, lax.full((16,), j, jnp.int32)],
                        vals,
                    )
                    return _

                lax.fori_loop(0, EMBEDDING_DIM, col_body, None)

        gather_desc(0, 0).start()
        gather_desc(1, 1).start()

        def win_body(it, _):
            w = it * 2
            for p in range(2):
                gather_desc(w + p, p).wait()

                @pl.when(it > 0)
                def _wait_prev():
                    scatter_desc(w + p, p).wait()

                select(w + p, p)

                @pl.when(w + p + 2 < _NWIN)
                def _next():
                    gather_desc(w + p + 2, p).start()

                scatter_desc(w + p, p).start()
            return _

        lax.fori_loop(0, _NWIN // 2, win_body, None)
        scatter_desc(_NWIN - 2, 0).wait()
        scatter_desc(_NWIN - 1, 1).wait()

    return gather_kernel


_gather = _make_gather()

_TBLK = 512
_TGRID = _PAIR_ROWS // _TBLK


def _make_transpose():
    def body(lo_ref, hi_ref, out_ref):
        lo_t = jnp.transpose(lo_ref[...], (1, 0))
        hi_t = jnp.transpose(hi_ref[...], (1, 0))
        out_ref[...] = jnp.concatenate([lo_t, hi_t], axis=1)

    return pl.pallas_call(
        body,
        grid=(_TGRID,),
        in_specs=[
            pl.BlockSpec((EMBEDDING_DIM, _TBLK), lambda g: (0, g)),
            pl.BlockSpec((EMBEDDING_DIM, _TBLK), lambda g: (0, g + _TGRID)),
        ],
        out_specs=pl.BlockSpec((_TBLK, _PW), lambda g: (g, 0)),
        out_shape=jax.ShapeDtypeStruct((_PAIR_ROWS, _PW), jnp.float32),
    )


_transpose_tc = _make_transpose()


def kernel(token_ids, embedding_matrix):
    table_t = embedding_matrix.T
    pairs = _transpose_tc(table_t, table_t)
    flat_ids = token_ids.reshape(_B).astype(jnp.int32)
    rows = _gather(pairs, flat_ids)
    return rows.reshape(BATCH, SEQ_LEN, EMBEDDING_DIM)


# speedup vs baseline: 1.8537x; 1.8537x over previous
import functools

import jax
import jax.numpy as jnp
from jax import lax
from jax.experimental import pallas as pl
from jax.experimental.pallas import tpu as pltpu, tpu_sc as plsc

NUM_EMBEDDINGS = 1000000
EMBEDDING_DIM = 64
BATCH = 4
SEQ_LEN = 8192

_INFO = plsc.get_sparse_core_info()
_NC, _NS = _INFO.num_cores, _INFO.num_subcores
_NW = _NC * _NS
_B = BATCH * SEQ_LEN
_B_PER_W = _B // _NW
_W = 128
_NWIN = _B_PER_W // _W
_PAIR_ROWS = 501760
_PW = 2 * EMBEDDING_DIM
_HALF = _PAIR_ROWS


def _make_gather():
    mesh = plsc.VectorSubcoreMesh(core_axis_name="c", subcore_axis_name="s")

    @functools.partial(
        pl.kernel,
        mesh=mesh,
        out_type=jax.ShapeDtypeStruct((_B, EMBEDDING_DIM), jnp.float32),
        scratch_types=[
            pltpu.VMEM((_B_PER_W,), jnp.int32),
            pltpu.VMEM((_B_PER_W,), jnp.int32),
            pltpu.VMEM((2, _W, _PW), jnp.float32),
            pltpu.VMEM((2, _W, EMBEDDING_DIM), jnp.float32),
            pltpu.SemaphoreType.DMA,
            pltpu.SemaphoreType.DMA,
            pltpu.SemaphoreType.DMA,
            pltpu.SemaphoreType.DMA,
        ],
        compiler_params=pltpu.CompilerParams(needs_layout_passes=False),
    )
    def gather_kernel(
        table_hbm, idx_hbm, out_hbm, idx_v, pair_v, rows_v, sel_v, g0, g1, s0, s1
    ):
        wid = lax.axis_index("s") * _NC + lax.axis_index("c")
        base = wid * _B_PER_W
        gsem = (g0, g1)
        ssem = (s0, s1)
        pltpu.sync_copy(idx_hbm.at[pl.ds(base, _B_PER_W)], idx_v)

        def pair_body(k, _):
            o = pl.multiple_of(k * 16, 16)
            ids = idx_v[pl.ds(o, 16)]
            hi = (ids >= _HALF).astype(jnp.int32)
            pair_v[pl.ds(o, 16)] = ids - hi * _HALF
            return _

        lax.fori_loop(0, _B_PER_W // 16, pair_body, None)

        def gather_desc(w, p):
            src = table_hbm.at[pair_v.at[pl.ds(pl.multiple_of(w * _W, _W), _W)]]
            return pltpu.make_async_copy(src, rows_v.at[p], gsem[p])

        def scatter_desc(w, p):
            dst = out_hbm.at[pl.ds(pl.multiple_of(base + w * _W, _W), _W)]
            return pltpu.make_async_copy(sel_v.at[p], dst, ssem[p])

        def select(w, p):
            for g in range(_W // 16):
                o = pl.multiple_of(w * _W + g * 16, 16)
                row16 = lax.iota(jnp.int32, 16) + g * 16
                off16 = (idx_v[pl.ds(o, 16)] >= _HALF).astype(
                    jnp.int32
                ) * EMBEDDING_DIM

                def col_body(j, _):
                    vals = plsc.load_gather(rows_v.at[p], [row16, off16 + j])
                    plsc.store_scatter(
                        sel_v.at[p],
                        [row16, lax.full((16,), j, jnp.int32)],
                        vals,
                    )
                    return _

                lax.fori_loop(0, EMBEDDING_DIM, col_body, None)

        gather_desc(0, 0).start()
        gather_desc(1, 1).start()

        def win_body(it, _):
            w = it * 2
            for p in range(2):
                gather_desc(w + p, p).wait()

                @pl.when(it > 0)
                def _wait_prev():
                    scatter_desc(w + p, p).wait()

                select(w + p, p)

                @pl.when(w + p + 2 < _NWIN)
                def _next():
                    gather_desc(w + p + 2, p).start()

                scatter_desc(w + p, p).start()
            return _

        lax.fori_loop(0, _NWIN // 2, win_body, None)
        scatter_desc(_NWIN - 2, 0).wait()
        scatter_desc(_NWIN - 1, 1).wait()

    return gather_kernel


_gather = _make_gather()

_TBLK = 2048
_TGRID = _PAIR_ROWS // _TBLK


def _make_transpose():
    def body(lo_ref, hi_ref, out_ref):
        lo_t = jnp.transpose(lo_ref[...], (1, 0))
        hi_t = jnp.transpose(hi_ref[...], (1, 0))
        out_ref[...] = jnp.concatenate([lo_t, hi_t], axis=1)

    return pl.pallas_call(
        body,
        grid=(_TGRID,),
        in_specs=[
            pl.BlockSpec((EMBEDDING_DIM, _TBLK), lambda g: (0, g)),
            pl.BlockSpec(
                (EMBEDDING_DIM, _TBLK),
                lambda g: (0, jnp.minimum(g + _TGRID, NUM_EMBEDDINGS // _TBLK)),
            ),
        ],
        out_specs=pl.BlockSpec((_TBLK, _PW), lambda g: (g, 0)),
        out_shape=jax.ShapeDtypeStruct((_PAIR_ROWS, _PW), jnp.float32),
    )


_transpose_tc = _make_transpose()


def kernel(token_ids, embedding_matrix):
    table_t = embedding_matrix.T
    pairs = _transpose_tc(table_t, table_t)
    flat_ids = token_ids.reshape(_B).astype(jnp.int32)
    rows = _gather(pairs, flat_ids)
    return rows.reshape(BATCH, SEQ_LEN, EMBEDDING_DIM)


# speedup vs baseline: 2.0518x; 1.1068x over previous
import functools

import jax
import jax.numpy as jnp
from jax import lax
from jax.experimental import pallas as pl
from jax.experimental.pallas import tpu as pltpu, tpu_sc as plsc

NUM_EMBEDDINGS = 1000000
EMBEDDING_DIM = 64
BATCH = 4
SEQ_LEN = 8192

_INFO = plsc.get_sparse_core_info()
_NC, _NS = _INFO.num_cores, _INFO.num_subcores
_NW = _NC * _NS
_B = BATCH * SEQ_LEN
_B_PER_W = _B // _NW
_W = 128
_NWIN = _B_PER_W // _W
_PAIR_ROWS = 501760
_PW = 2 * EMBEDDING_DIM
_HALF = _PAIR_ROWS


def _make_gather():
    mesh = plsc.VectorSubcoreMesh(core_axis_name="c", subcore_axis_name="s")

    @functools.partial(
        pl.kernel,
        mesh=mesh,
        out_type=jax.ShapeDtypeStruct((_B, EMBEDDING_DIM), jnp.float32),
        scratch_types=[
            pltpu.VMEM((_B_PER_W,), jnp.int32),
            pltpu.VMEM((_B_PER_W,), jnp.int32),
            pltpu.VMEM((2, _W, _PW), jnp.float32),
            pltpu.VMEM((2, _W, EMBEDDING_DIM), jnp.float32),
            pltpu.SemaphoreType.DMA,
            pltpu.SemaphoreType.DMA,
            pltpu.SemaphoreType.DMA,
            pltpu.SemaphoreType.DMA,
        ],
        compiler_params=pltpu.CompilerParams(needs_layout_passes=False),
    )
    def gather_kernel(
        table_hbm, idx_hbm, out_hbm, idx_v, pair_v, rows_v, sel_v, g0, g1, s0, s1
    ):
        wid = lax.axis_index("s") * _NC + lax.axis_index("c")
        base = wid * _B_PER_W
        gsem = (g0, g1)
        ssem = (s0, s1)
        pltpu.sync_copy(idx_hbm.at[pl.ds(base, _B_PER_W)], idx_v)

        def pair_body(k, _):
            o = pl.multiple_of(k * 16, 16)
            ids = idx_v[pl.ds(o, 16)]
            hi = (ids >= _HALF).astype(jnp.int32)
            pair_v[pl.ds(o, 16)] = ids - hi * _HALF
            return _

        lax.fori_loop(0, _B_PER_W // 16, pair_body, None)

        def gather_desc(w, p):
            src = table_hbm.at[pair_v.at[pl.ds(pl.multiple_of(w * _W, _W), _W)]]
            return pltpu.make_async_copy(src, rows_v.at[p], gsem[p])

        def scatter_desc(w, p):
            dst = out_hbm.at[pl.ds(pl.multiple_of(base + w * _W, _W), _W)]
            return pltpu.make_async_copy(sel_v.at[p], dst, ssem[p])

        def select(w, p):
            for g in range(_W // 16):
                o = pl.multiple_of(w * _W + g * 16, 16)
                row16 = lax.iota(jnp.int32, 16) + g * 16
                off16 = (idx_v[pl.ds(o, 16)] >= _HALF).astype(
                    jnp.int32
                ) * EMBEDDING_DIM

                def col_body(j, _):
                    vals = plsc.load_gather(rows_v.at[p], [row16, off16 + j])
                    plsc.store_scatter(
                        sel_v.at[p],
                        [row16, lax.full((16,), j, jnp.int32)],
                        vals,
                    )
                    return _

                lax.fori_loop(0, EMBEDDING_DIM, col_body, None)

        gather_desc(0, 0).start()
        gather_desc(1, 1).start()

        def win_body(it, _):
            w = it * 2
            for p in range(2):
                gather_desc(w + p, p).wait()

                @pl.when(it > 0)
                def _wait_prev():
                    scatter_desc(w + p, p).wait()

                select(w + p, p)

                @pl.when(w + p + 2 < _NWIN)
                def _next():
                    gather_desc(w + p + 2, p).start()

                scatter_desc(w + p, p).start()
            return _

        lax.fori_loop(0, _NWIN // 2, win_body, None)
        scatter_desc(_NWIN - 2, 0).wait()
        scatter_desc(_NWIN - 1, 1).wait()

    return gather_kernel


_gather = _make_gather()

_TBLK = 2048
_TGRID = _PAIR_ROWS // _TBLK


def _make_transpose():
    def body(lo_ref, hi_ref, out_ref):
        i0 = lax.broadcasted_iota(jnp.int32, (128, 128), 0)
        i1 = lax.broadcasted_iota(jnp.int32, (128, 128), 1)
        ident = (i0 == i1).astype(jnp.float32)
        dn = (((1,), (1,)), ((), ()))
        for j in range(_TBLK // 128):
            sl = pl.ds(j * 128, 128)
            lo_t = lax.dot_general(
                ident, lo_ref[:, sl], dn, preferred_element_type=jnp.float32
            )
            hi_t = lax.dot_general(
                ident, hi_ref[:, sl], dn, preferred_element_type=jnp.float32
            )
            out_ref[sl, :] = jnp.concatenate([lo_t, hi_t], axis=1)

    return pl.pallas_call(
        body,
        grid=(_TGRID,),
        in_specs=[
            pl.BlockSpec((EMBEDDING_DIM, _TBLK), lambda g: (0, g)),
            pl.BlockSpec(
                (EMBEDDING_DIM, _TBLK),
                lambda g: (0, jnp.minimum(g + _TGRID, NUM_EMBEDDINGS // _TBLK)),
            ),
        ],
        out_specs=pl.BlockSpec((_TBLK, _PW), lambda g: (g, 0)),
        out_shape=jax.ShapeDtypeStruct((_PAIR_ROWS, _PW), jnp.float32),
    )


_transpose_tc = _make_transpose()


def kernel(token_ids, embedding_matrix):
    table_t = embedding_matrix.T
    pairs = _transpose_tc(table_t, table_t)
    flat_ids = token_ids.reshape(_B).astype(jnp.int32)
    rows = _gather(pairs, flat_ids)
    return rows.reshape(BATCH, SEQ_LEN, EMBEDDING_DIM)


# speedup vs baseline: 2.0547x; 1.0014x over previous
import functools

import jax
import jax.numpy as jnp
from jax import lax
from jax.experimental import pallas as pl
from jax.experimental.pallas import tpu as pltpu, tpu_sc as plsc

NUM_EMBEDDINGS = 1000000
EMBEDDING_DIM = 64
BATCH = 4
SEQ_LEN = 8192

_INFO = plsc.get_sparse_core_info()
_NC, _NS = _INFO.num_cores, _INFO.num_subcores
_NW = _NC * _NS
_B = BATCH * SEQ_LEN
_B_PER_W = _B // _NW
_W = 128
_NWIN = _B_PER_W // _W
_PAIR_ROWS = 501760
_PW = 2 * EMBEDDING_DIM
_HALF = _PAIR_ROWS


def _make_gather():
    mesh = plsc.VectorSubcoreMesh(core_axis_name="c", subcore_axis_name="s")

    @functools.partial(
        pl.kernel,
        mesh=mesh,
        out_type=jax.ShapeDtypeStruct((_B, EMBEDDING_DIM), jnp.float32),
        scratch_types=[
            pltpu.VMEM((_B_PER_W,), jnp.int32),
            pltpu.VMEM((_B_PER_W,), jnp.int32),
            pltpu.VMEM((2, _W, _PW), jnp.float32),
            pltpu.VMEM((2, _W, EMBEDDING_DIM), jnp.float32),
            pltpu.SemaphoreType.DMA,
            pltpu.SemaphoreType.DMA,
            pltpu.SemaphoreType.DMA,
            pltpu.SemaphoreType.DMA,
        ],
        compiler_params=pltpu.CompilerParams(needs_layout_passes=False),
    )
    def gather_kernel(
        table_hbm, idx_hbm, out_hbm, idx_v, pair_v, rows_v, sel_v, g0, g1, s0, s1
    ):
        wid = lax.axis_index("s") * _NC + lax.axis_index("c")
        base = wid * _B_PER_W
        gsem = (g0, g1)
        ssem = (s0, s1)
        pltpu.sync_copy(idx_hbm.at[pl.ds(base, _B_PER_W)], idx_v)

        def pair_body(k, _):
            o = pl.multiple_of(k * 16, 16)
            ids = idx_v[pl.ds(o, 16)]
            hi = (ids >= _HALF).astype(jnp.int32)
            pair_v[pl.ds(o, 16)] = ids - hi * _HALF
            return _

        lax.fori_loop(0, _B_PER_W // 16, pair_body, None)

        def gather_desc(w, p):
            src = table_hbm.at[pair_v.at[pl.ds(pl.multiple_of(w * _W, _W), _W)]]
            return pltpu.make_async_copy(src, rows_v.at[p], gsem[p])

        def scatter_desc(w, p):
            dst = out_hbm.at[pl.ds(pl.multiple_of(base + w * _W, _W), _W)]
            return pltpu.make_async_copy(sel_v.at[p], dst, ssem[p])

        def select(w, p):
            for g in range(_W // 16):
                o = pl.multiple_of(w * _W + g * 16, 16)
                row16 = lax.iota(jnp.int32, 16) + g * 16
                off16 = (idx_v[pl.ds(o, 16)] >= _HALF).astype(
                    jnp.int32
                ) * EMBEDDING_DIM

                def col_body(j, _):
                    vals = plsc.load_gather(rows_v.at[p], [row16, off16 + j])
                    plsc.store_scatter(
                        sel_v.at[p],
                        [row16, lax.full((16,), j, jnp.int32)],
                        vals,
                    )
                    return _

                lax.fori_loop(0, EMBEDDING_DIM, col_body, None)

        gather_desc(0, 0).start()
        gather_desc(1, 1).start()

        def win_body(it, _):
            w = it * 2
            for p in range(2):
                gather_desc(w + p, p).wait()

                @pl.when(it > 0)
                def _wait_prev():
                    scatter_desc(w + p, p).wait()

                select(w + p, p)

                @pl.when(w + p + 2 < _NWIN)
                def _next():
                    gather_desc(w + p + 2, p).start()

                scatter_desc(w + p, p).start()
            return _

        lax.fori_loop(0, _NWIN // 2, win_body, None)
        scatter_desc(_NWIN - 2, 0).wait()
        scatter_desc(_NWIN - 1, 1).wait()

    return gather_kernel


_gather = _make_gather()

_TBLK = 2048
_TGRID = _PAIR_ROWS // _TBLK


def _make_transpose():
    def body(lo_ref, hi_ref, out_ref):
        i0 = lax.broadcasted_iota(jnp.int32, (128, 128), 0)
        i1 = lax.broadcasted_iota(jnp.int32, (128, 128), 1)
        ident = (i0 == i1).astype(jnp.float32)
        dn = (((1,), (1,)), ((), ()))
        for j in range(_TBLK // 128):
            sl = pl.ds(j * 128, 128)
            x2 = jnp.concatenate([lo_ref[:, sl], hi_ref[:, sl]], axis=0)
            out_ref[sl, :] = lax.dot_general(
                ident, x2, dn, preferred_element_type=jnp.float32
            )

    return pl.pallas_call(
        body,
        grid=(_TGRID,),
        in_specs=[
            pl.BlockSpec((EMBEDDING_DIM, _TBLK), lambda g: (0, g)),
            pl.BlockSpec(
                (EMBEDDING_DIM, _TBLK),
                lambda g: (0, jnp.minimum(g + _TGRID, NUM_EMBEDDINGS // _TBLK)),
            ),
        ],
        out_specs=pl.BlockSpec((_TBLK, _PW), lambda g: (g, 0)),
        out_shape=jax.ShapeDtypeStruct((_PAIR_ROWS, _PW), jnp.float32),
    )


_transpose_tc = _make_transpose()


def kernel(token_ids, embedding_matrix):
    table_t = embedding_matrix.T
    pairs = _transpose_tc(table_t, table_t)
    flat_ids = token_ids.reshape(_B).astype(jnp.int32)
    rows = _gather(pairs, flat_ids)
    return rows.reshape(BATCH, SEQ_LEN, EMBEDDING_DIM)


# speedup vs baseline: 2.5382x; 1.2354x over previous
import functools

import jax
import jax.numpy as jnp
from jax import lax
from jax.experimental import pallas as pl
from jax.experimental.pallas import tpu as pltpu, tpu_sc as plsc

NUM_EMBEDDINGS = 1000000
EMBEDDING_DIM = 64
BATCH = 4
SEQ_LEN = 8192

_INFO = plsc.get_sparse_core_info()
_NC, _NS = _INFO.num_cores, _INFO.num_subcores
_NW = _NC * _NS
_B = BATCH * SEQ_LEN
_B_PER_W = _B // _NW
_W = 128
_NWIN = _B_PER_W // _W
_PAIR_ROWS = 503808
_PW = 2 * EMBEDDING_DIM
_HALF = _PAIR_ROWS


def _make_gather():
    mesh = plsc.VectorSubcoreMesh(core_axis_name="c", subcore_axis_name="s")

    @functools.partial(
        pl.kernel,
        mesh=mesh,
        out_type=jax.ShapeDtypeStruct((_B, EMBEDDING_DIM), jnp.float32),
        scratch_types=[
            pltpu.VMEM((_B_PER_W,), jnp.int32),
            pltpu.VMEM((_B_PER_W,), jnp.int32),
            pltpu.VMEM((2, _W, _PW), jnp.float32),
            pltpu.VMEM((2, _W, EMBEDDING_DIM), jnp.float32),
            pltpu.SemaphoreType.DMA,
            pltpu.SemaphoreType.DMA,
            pltpu.SemaphoreType.DMA,
            pltpu.SemaphoreType.DMA,
        ],
        compiler_params=pltpu.CompilerParams(needs_layout_passes=False),
    )
    def gather_kernel(
        table_hbm, idx_hbm, out_hbm, idx_v, pair_v, rows_v, sel_v, g0, g1, s0, s1
    ):
        wid = lax.axis_index("s") * _NC + lax.axis_index("c")
        base = wid * _B_PER_W
        gsem = (g0, g1)
        ssem = (s0, s1)
        pltpu.sync_copy(idx_hbm.at[pl.ds(base, _B_PER_W)], idx_v)

        def pair_body(k, _):
            o = pl.multiple_of(k * 16, 16)
            ids = idx_v[pl.ds(o, 16)]
            hi = (ids >= _HALF).astype(jnp.int32)
            pair_v[pl.ds(o, 16)] = ids - hi * _HALF
            return _

        lax.fori_loop(0, _B_PER_W // 16, pair_body, None)

        def gather_desc(w, p):
            src = table_hbm.at[pair_v.at[pl.ds(pl.multiple_of(w * _W, _W), _W)]]
            return pltpu.make_async_copy(src, rows_v.at[p], gsem[p])

        def scatter_desc(w, p):
            dst = out_hbm.at[pl.ds(pl.multiple_of(base + w * _W, _W), _W)]
            return pltpu.make_async_copy(sel_v.at[p], dst, ssem[p])

        def select(w, p):
            for g in range(_W // 16):
                o = pl.multiple_of(w * _W + g * 16, 16)
                row16 = lax.iota(jnp.int32, 16) + g * 16
                off16 = (idx_v[pl.ds(o, 16)] >= _HALF).astype(
                    jnp.int32
                ) * EMBEDDING_DIM

                def col_body(j, _):
                    vals = plsc.load_gather(rows_v.at[p], [row16, off16 + j])
                    plsc.store_scatter(
                        sel_v.at[p],
                        [row16, lax.full((16,), j, jnp.int32)],
                        vals,
                    )
                    return _

                lax.fori_loop(0, EMBEDDING_DIM, col_body, None)

        gather_desc(0, 0).start()
        gather_desc(1, 1).start()

        def win_body(it, _):
            w = it * 2
            for p in range(2):
                gather_desc(w + p, p).wait()

                @pl.when(it > 0)
                def _wait_prev():
                    scatter_desc(w + p, p).wait()

                select(w + p, p)

                @pl.when(w + p + 2 < _NWIN)
                def _next():
                    gather_desc(w + p + 2, p).start()

                scatter_desc(w + p, p).start()
            return _

        lax.fori_loop(0, _NWIN // 2, win_body, None)
        scatter_desc(_NWIN - 2, 0).wait()
        scatter_desc(_NWIN - 1, 1).wait()

    return gather_kernel


_gather = _make_gather()

_TBLK = 4096
_TGRID = _PAIR_ROWS // _TBLK


def _make_transpose():
    def body(lo_ref, hi_ref, out_ref):
        i0 = lax.broadcasted_iota(jnp.int32, (128, 128), 0)
        i1 = lax.broadcasted_iota(jnp.int32, (128, 128), 1)
        ident = (i0 == i1).astype(jnp.float32)
        dn = (((1,), (1,)), ((), ()))
        for j in range(_TBLK // 128):
            sl = pl.ds(j * 128, 128)
            x2 = jnp.concatenate([lo_ref[:, sl], hi_ref[:, sl]], axis=0)
            out_ref[sl, :] = lax.dot_general(
                ident, x2, dn, preferred_element_type=jnp.float32
            )

    return pl.pallas_call(
        body,
        grid=(_TGRID,),
        in_specs=[
            pl.BlockSpec((EMBEDDING_DIM, _TBLK), lambda g: (0, g)),
            pl.BlockSpec(
                (EMBEDDING_DIM, _TBLK),
                lambda g: (0, jnp.minimum(g + _TGRID, NUM_EMBEDDINGS // _TBLK)),
            ),
        ],
        out_specs=pl.BlockSpec((_TBLK, _PW), lambda g: (g, 0)),
        out_shape=jax.ShapeDtypeStruct((_PAIR_ROWS, _PW), jnp.float32),
    )


_transpose_tc = _make_transpose()


def kernel(token_ids, embedding_matrix):
    table_t = embedding_matrix.T
    pairs = _transpose_tc(table_t, table_t)
    flat_ids = token_ids.reshape(_B).astype(jnp.int32)
    rows = _gather(pairs, flat_ids)
    return rows.reshape(BATCH, SEQ_LEN, EMBEDDING_DIM)
